# Initial kernel scaffold; baseline (speedup 1.0000x reference)
import jax
import jax.numpy as jnp
from jax.experimental import pallas as pl


def kernel(node_feats, edge_index, node_graph_ids, solv_node_feats, solv_edge_index, solv_node_graph_ids, W_emb, b_emb, W_emb_s, b_emb_s, W_gcn, b_gcn, W_gcn_s, b_gcn_s, W_l1, b_l1, W_l2, b_l2, W_l3a, b_l3a, W_l3b, b_l3b, W_out, b_out):
    raise NotImplementedError("write your pallas kernel here")


# speedup vs baseline: 4.5297x; 4.5297x over previous
import functools

import jax
import jax.numpy as jnp
from jax import lax
from jax.experimental import pallas as pl
from jax.experimental.pallas import tpu as pltpu
from jax.experimental.pallas import tpu_sc as plsc

N = 10000
E = 160000
NG = 128
NTILES = 16
NW = 32
EPT = E // NTILES
EPW = E // NW
CH = 128
NCH = 79
NCHS = 40
PC = 128
TRASH = N
ACC_ROWS = 10112
STR = 632
STR_LAST = N - 15 * STR
PCH = 5
PTRASH = NG
PACC_ROWS = 144


def _sc_mesh():
    return plsc.VectorSubcoreMesh(core_axis_name="c", subcore_axis_name="s")


def _striped_copy(src, dst, sid):
    @pl.when(sid < 15)
    def _():
        pltpu.sync_copy(src.at[pl.ds(sid * STR, STR)],
                        dst.at[pl.ds(sid * STR, STR)])

    @pl.when(sid == 15)
    def _():
        pltpu.sync_copy(src.at[pl.ds(15 * STR, STR_LAST)],
                        dst.at[pl.ds(15 * STR, STR_LAST)])


def _zero_rows(zbuf, rows, width):
    z = jnp.zeros((16,), jnp.float32)

    def body(r, _):
        for c in range(width // 16):
            zbuf[r, pl.ds(c * 16, 16)] = z
        return 0

    lax.fori_loop(0, rows, body, 0)


def _scatter_loop(table, srcv, dstv, buf, gs, acc, nch):
    def body(j, _):
        pltpu.make_async_copy(table.at[srcv.at[j]], buf, gs).start()
        pltpu.make_async_copy(table.at[srcv.at[j]], buf, gs).wait()
        pltpu.sync_copy(buf, acc.at[dstv.at[j]], add=True)
        return 0

    lax.fori_loop(0, nch, body, 0)


@functools.partial(
    pl.kernel,
    out_type=[
        jax.ShapeDtypeStruct((N, 128), jnp.float32),
        jax.ShapeDtypeStruct((N, 128), jnp.float32),
    ],
    mesh=_sc_mesh(),
    scratch_types=[
        pltpu.VMEM((NCH, CH), jnp.int32),
        pltpu.VMEM((NCH, CH), jnp.int32),
        pltpu.VMEM((CH, 128), jnp.float32),
        pltpu.VMEM_SHARED((ACC_ROWS, 128), jnp.float32),
        pltpu.SemaphoreType.DMA,
    ],
)
def _sc_scatter_main(hs_lo, hs_hi, srcp, dstp, zeros_hbm, out_lo, out_hi,
                     srcv, dstv, buf, acc, gs):
    cid = lax.axis_index("c")
    sid = lax.axis_index("s")
    pltpu.sync_copy(srcp.at[sid], srcv)
    pltpu.sync_copy(dstp.at[sid], dstv)
    pltpu.sync_copy(zeros_hbm.at[pl.ds(sid * STR, STR)],
                    acc.at[pl.ds(sid * STR, STR)])
    plsc.subcore_barrier()

    def run(table, out):
        _scatter_loop(table, srcv, dstv, buf, gs, acc, NCH)
        plsc.subcore_barrier()
        _striped_copy(acc, out, sid)

    @pl.when(cid == 0)
    def _():
        run(hs_lo, out_lo)

    @pl.when(cid == 1)
    def _():
        run(hs_hi, out_hi)


@functools.partial(
    pl.kernel,
    out_type=[
        jax.ShapeDtypeStruct((N, 128), jnp.float32),
        jax.ShapeDtypeStruct((N, 128), jnp.float32),
    ],
    mesh=_sc_mesh(),
    scratch_types=[
        pltpu.VMEM((NCHS, CH), jnp.int32),
        pltpu.VMEM((NCHS, CH), jnp.int32),
        pltpu.VMEM((CH, 128), jnp.float32),
        pltpu.VMEM_SHARED((ACC_ROWS, 128), jnp.float32),
        pltpu.SemaphoreType.DMA,
    ],
)
def _sc_scatter_solv(hs_pad, srcp, dstp, zeros_hbm, out0, out1,
                     srcv, dstv, buf, acc, gs):
    cid = lax.axis_index("c")
    sid = lax.axis_index("s")
    wid = cid * NTILES + sid
    pltpu.sync_copy(srcp.at[wid], srcv)
    pltpu.sync_copy(dstp.at[wid], dstv)
    pltpu.sync_copy(zeros_hbm.at[pl.ds(sid * STR, STR)],
                    acc.at[pl.ds(sid * STR, STR)])
    plsc.subcore_barrier()
    _scatter_loop(hs_pad, srcv, dstv, buf, gs, acc, NCHS)
    plsc.subcore_barrier()

    @pl.when(cid == 0)
    def _():
        _striped_copy(acc, out0, sid)

    @pl.when(cid == 1)
    def _():
        _striped_copy(acc, out1, sid)


@functools.partial(
    pl.kernel,
    out_type=[
        jax.ShapeDtypeStruct((N, 128), jnp.float32),
        jax.ShapeDtypeStruct((N, 128), jnp.float32),
    ],
    mesh=_sc_mesh(),
    scratch_types=[
        pltpu.VMEM((NCHS, CH), jnp.int32),
        pltpu.VMEM((CH, 128), jnp.float32),
        pltpu.VMEM_SHARED((ACC_ROWS, 128), jnp.float32),
    ],
)
def _sc_deg(dstp, ones_hbm, zeros_hbm, out0, out1, dstv, ones_buf, acc):
    cid = lax.axis_index("c")
    sid = lax.axis_index("s")
    wid = cid * NTILES + sid
    pltpu.sync_copy(dstp.at[wid], dstv)
    pltpu.sync_copy(ones_hbm, ones_buf)
    pltpu.sync_copy(zeros_hbm.at[pl.ds(sid * STR, STR)],
                    acc.at[pl.ds(sid * STR, STR)])
    plsc.subcore_barrier()

    def body(j, _):
        pltpu.sync_copy(ones_buf, acc.at[dstv.at[j]], add=True)
        return 0

    lax.fori_loop(0, NCHS, body, 0)
    plsc.subcore_barrier()

    @pl.when(cid == 0)
    def _():
        _striped_copy(acc, out0, sid)

    @pl.when(cid == 1)
    def _():
        _striped_copy(acc, out1, sid)


def _pool_run(table, out, acc, gidv, buf, sid, drain):
    base = sid * STR
    for j in range(PCH - 1):
        pltpu.sync_copy(table.at[pl.ds(base + j * PC, PC)], buf)
        pltpu.sync_copy(buf, acc.at[gidv.at[j]], add=True)

    @pl.when(sid < 15)
    def _():
        pltpu.sync_copy(table.at[pl.ds(base + (PCH - 1) * PC, STR - (PCH - 1) * PC)],
                        buf.at[pl.ds(0, STR - (PCH - 1) * PC)])

    @pl.when(sid == 15)
    def _():
        pltpu.sync_copy(table.at[pl.ds(15 * STR + (PCH - 1) * PC, STR_LAST - (PCH - 1) * PC)],
                        buf.at[pl.ds(0, STR_LAST - (PCH - 1) * PC)])

    pltpu.sync_copy(buf, acc.at[gidv.at[PCH - 1]], add=True)
    plsc.subcore_barrier()

    @pl.when(drain)
    def _():
        pltpu.sync_copy(acc.at[pl.ds(0, NG)], out)


def _make_sc_pool(two_tables):
    n_out = 2 if two_tables else 1

    @functools.partial(
        pl.kernel,
        out_type=[jax.ShapeDtypeStruct((NG, 128), jnp.float32)] * n_out,
        mesh=_sc_mesh(),
        scratch_types=[
            pltpu.VMEM((PCH, PC), jnp.int32),
            pltpu.VMEM((PC, 128), jnp.float32),
            pltpu.VMEM((PACC_ROWS, 128), jnp.float32),
            pltpu.VMEM_SHARED((PACC_ROWS, 128), jnp.float32),
        ],
    )
    def sc_pool(*args):
        if two_tables:
            h_lo, h_hi, gidp, out_lo, out_hi, gidv, buf, zbuf, acc = args
        else:
            h_pad, gidp, out, gidv, buf, zbuf, acc = args
        cid = lax.axis_index("c")
        sid = lax.axis_index("s")
        pltpu.sync_copy(gidp.at[sid], gidv)

        @pl.when(sid == 0)
        def _():
            _zero_rows(zbuf, PACC_ROWS, 128)
            pltpu.sync_copy(zbuf, acc)

        plsc.subcore_barrier()

        if two_tables:
            @pl.when(cid == 0)
            def _():
                _pool_run(h_lo, out_lo, acc, gidv, buf, sid, sid == 0)

            @pl.when(cid == 1)
            def _():
                _pool_run(h_hi, out_hi, acc, gidv, buf, sid, sid == 0)
        else:
            _pool_run(h_pad, out, acc, gidv, buf, sid,
                      jnp.logical_and(cid == 0, sid == 0))

    return sc_pool


_sc_pool_main = _make_sc_pool(True)
_sc_pool_solv = _make_sc_pool(False)


_RB = 1000


def _dinv_of(d0_ref, d1_ref):
    d = d0_ref[:, 0:1] + d1_ref[:, 0:1]
    return lax.rsqrt(jnp.maximum(d, 1.0))


def _emb_main_body(x_ref, w_ref, b_ref, d0_ref, d1_ref,
                   h_ref, lo_ref, hi_ref, dv_ref):
    h = jnp.dot(x_ref[...], w_ref[...], preferred_element_type=jnp.float32)
    h = h + b_ref[...]
    dinv = _dinv_of(d0_ref, d1_ref)
    hs = h * dinv
    h_ref[...] = h
    lo_ref[...] = hs[:, :128]
    hi_ref[...] = hs[:, 128:]
    dv_ref[...] = jnp.broadcast_to(dinv, (dinv.shape[0], 16))


_tc_emb_main = pl.pallas_call(
    _emb_main_body,
    grid=(N // _RB,),
    in_specs=[
        pl.BlockSpec((_RB, 256), lambda i: (i, 0)),
        pl.BlockSpec((256, 256), lambda i: (0, 0)),
        pl.BlockSpec((1, 256), lambda i: (0, 0)),
        pl.BlockSpec((_RB, 128), lambda i: (i, 0)),
        pl.BlockSpec((_RB, 128), lambda i: (i, 0)),
    ],
    out_specs=[
        pl.BlockSpec((_RB, 256), lambda i: (i, 0)),
        pl.BlockSpec((_RB, 128), lambda i: (i, 0)),
        pl.BlockSpec((_RB, 128), lambda i: (i, 0)),
        pl.BlockSpec((_RB, 16), lambda i: (i, 0)),
    ],
    out_shape=[
        jax.ShapeDtypeStruct((N, 256), jnp.float32),
        jax.ShapeDtypeStruct((N, 128), jnp.float32),
        jax.ShapeDtypeStruct((N, 128), jnp.float32),
        jax.ShapeDtypeStruct((N, 16), jnp.float32),
    ],
)


def _layer_main_body(alo_ref, ahi_ref, h_ref, w_ref, b_ref, dv_ref,
                     h_out_ref, lo_ref, hi_ref, *, scale_out):
    dinv = dv_ref[:, 0:1]
    alo = alo_ref[...] * dinv
    ahi = ahi_ref[...] * dinv
    out = jnp.dot(alo, w_ref[:128, :], preferred_element_type=jnp.float32)
    out += jnp.dot(ahi, w_ref[128:, :], preferred_element_type=jnp.float32)
    out += b_ref[...]
    h_new = jnp.maximum(out, 0.0) + h_ref[...]
    h_out_ref[...] = h_new
    hs = h_new * dinv if scale_out else h_new
    lo_ref[...] = hs[:, :128]
    hi_ref[...] = hs[:, 128:]


def _make_tc_layer_main(scale_out):
    return pl.pallas_call(
        functools.partial(_layer_main_body, scale_out=scale_out),
        grid=(N // _RB,),
        in_specs=[
            pl.BlockSpec((_RB, 128), lambda i: (i, 0)),
            pl.BlockSpec((_RB, 128), lambda i: (i, 0)),
            pl.BlockSpec((_RB, 256), lambda i: (i, 0)),
            pl.BlockSpec((256, 256), lambda i: (0, 0)),
            pl.BlockSpec((1, 256), lambda i: (0, 0)),
            pl.BlockSpec((_RB, 16), lambda i: (i, 0)),
        ],
        out_specs=[
            pl.BlockSpec((_RB, 256), lambda i: (i, 0)),
            pl.BlockSpec((_RB, 128), lambda i: (i, 0)),
            pl.BlockSpec((_RB, 128), lambda i: (i, 0)),
        ],
        out_shape=[
            jax.ShapeDtypeStruct((N, 256), jnp.float32),
            jax.ShapeDtypeStruct((N, 128), jnp.float32),
            jax.ShapeDtypeStruct((N, 128), jnp.float32),
        ],
    )


def _emb_solv_body(x_ref, w_ref, b_ref, d0_ref, d1_ref, h_ref, hs_ref, dv_ref):
    h = jnp.dot(x_ref[...], w_ref[...], preferred_element_type=jnp.float32)
    h = h + b_ref[...]
    dinv = _dinv_of(d0_ref, d1_ref)
    hs = h * dinv
    h_ref[...] = h
    hs_ref[...] = jnp.concatenate(
        [hs, jnp.zeros((hs.shape[0], 64), jnp.float32)], axis=1)
    dv_ref[...] = jnp.broadcast_to(dinv, (dinv.shape[0], 16))


_tc_emb_solv = pl.pallas_call(
    _emb_solv_body,
    grid=(N // _RB,),
    in_specs=[
        pl.BlockSpec((_RB, 256), lambda i: (i, 0)),
        pl.BlockSpec((256, 64), lambda i: (0, 0)),
        pl.BlockSpec((1, 64), lambda i: (0, 0)),
        pl.BlockSpec((_RB, 128), lambda i: (i, 0)),
        pl.BlockSpec((_RB, 128), lambda i: (i, 0)),
    ],
    out_specs=[
        pl.BlockSpec((_RB, 64), lambda i: (i, 0)),
        pl.BlockSpec((_RB, 128), lambda i: (i, 0)),
        pl.BlockSpec((_RB, 16), lambda i: (i, 0)),
    ],
    out_shape=[
        jax.ShapeDtypeStruct((N, 64), jnp.float32),
        jax.ShapeDtypeStruct((N, 128), jnp.float32),
        jax.ShapeDtypeStruct((N, 16), jnp.float32),
    ],
)


def _layer_solv_body(a0_ref, a1_ref, h_ref, w_ref, b_ref, dv_ref,
                     h_out_ref, hs_ref, *, scale_out):
    dinv = dv_ref[:, 0:1]
    a = (a0_ref[:, :64] + a1_ref[:, :64]) * dinv
    out = jnp.dot(a, w_ref[...], preferred_element_type=jnp.float32)
    out += b_ref[...]
    h_new = jnp.maximum(out, 0.0) + h_ref[...]
    h_out_ref[...] = h_new
    hs = h_new * dinv if scale_out else h_new
    hs_ref[...] = jnp.concatenate(
        [hs, jnp.zeros((hs.shape[0], 64), jnp.float32)], axis=1)


def _make_tc_layer_solv(scale_out):
    return pl.pallas_call(
        functools.partial(_layer_solv_body, scale_out=scale_out),
        grid=(N // _RB,),
        in_specs=[
            pl.BlockSpec((_RB, 128), lambda i: (i, 0)),
            pl.BlockSpec((_RB, 128), lambda i: (i, 0)),
            pl.BlockSpec((_RB, 64), lambda i: (i, 0)),
            pl.BlockSpec((64, 64), lambda i: (0, 0)),
            pl.BlockSpec((1, 64), lambda i: (0, 0)),
            pl.BlockSpec((_RB, 16), lambda i: (i, 0)),
        ],
        out_specs=[
            pl.BlockSpec((_RB, 64), lambda i: (i, 0)),
            pl.BlockSpec((_RB, 128), lambda i: (i, 0)),
        ],
        out_shape=[
            jax.ShapeDtypeStruct((N, 64), jnp.float32),
            jax.ShapeDtypeStruct((N, 128), jnp.float32),
        ],
    )


def _head_body(hg_lo, hg_hi, hsg_ref,
               w1_ref, b1_ref, w2_ref, b2_ref,
               w3a_ref, b3a_ref, w3b_ref, b3b_ref, wo_ref, bo_ref, out_ref):
    hg = jnp.concatenate([hg_lo[...], hg_hi[...]], axis=1)
    hsg = hsg_ref[:, :64]
    for i in range(2):
        hg = jnp.maximum(
            jnp.dot(hg, w1_ref[i], preferred_element_type=jnp.float32)
            + b1_ref[i], 0.0)
    for i in range(2):
        hsg = jnp.maximum(
            jnp.dot(hsg, w2_ref[i], preferred_element_type=jnp.float32)
            + b2_ref[i], 0.0)
    hc = jnp.concatenate([hg, hsg], axis=1)
    hc = jnp.maximum(
        jnp.dot(hc, w3a_ref[...], preferred_element_type=jnp.float32)
        + b3a_ref[...], 0.0)
    hc = jnp.maximum(
        jnp.dot(hc, w3b_ref[...], preferred_element_type=jnp.float32)
        + b3b_ref[...], 0.0)
    out_ref[...] = (
        jnp.dot(hc, wo_ref[...], preferred_element_type=jnp.float32)
        + bo_ref[...])


_tc_head = pl.pallas_call(
    _head_body,
    out_shape=jax.ShapeDtypeStruct((NG, 1), jnp.float32),
)


def _pad_edges(edge_index, n_workers, nch):
    ept = E // n_workers
    src = edge_index[0].astype(jnp.int32).reshape(n_workers, ept)
    dst = edge_index[1].astype(jnp.int32).reshape(n_workers, ept)
    pad = nch * CH - ept
    srcp = jnp.concatenate(
        [src, jnp.zeros((n_workers, pad), jnp.int32)], axis=1
    ).reshape(n_workers, nch, CH)
    dstp = jnp.concatenate(
        [dst, jnp.full((n_workers, pad), TRASH, jnp.int32)], axis=1
    ).reshape(n_workers, nch, CH)
    return srcp, dstp


def _pad_gids(gids):
    g = gids.astype(jnp.int32)
    slots = PCH * PC
    pos = jnp.arange(NTILES)[:, None] * STR + jnp.arange(slots)[None, :]
    lens = jnp.where(jnp.arange(NTILES) < 15, STR, STR_LAST)[:, None]
    valid = jnp.arange(slots)[None, :] < lens
    gp = jnp.where(valid, g[jnp.clip(pos, 0, N - 1)], PTRASH)
    return gp.reshape(NTILES, PCH, PC)


_tc_layer_main = _make_tc_layer_main(True)
_tc_layer_main_last = _make_tc_layer_main(False)
_tc_layer_solv = _make_tc_layer_solv(True)
_tc_layer_solv_last = _make_tc_layer_solv(False)


def _main_branch(x, edge_index, gids, w_emb, b_emb, w_gcn, b_gcn, ones128, zeros128):
    srcp, dstp = _pad_edges(edge_index, NTILES, NCH)
    dstp2 = _pad_edges(edge_index, NW, NCHS)[1]
    gidp = _pad_gids(gids)
    d0, d1 = _sc_deg(dstp2, ones128, zeros128)
    h, hs_lo, hs_hi, dv = _tc_emb_main(x, w_emb, b_emb.reshape(1, -1), d0, d1)
    nl = w_gcn.shape[0]
    for i in range(nl):
        agg_lo, agg_hi = _sc_scatter_main(hs_lo, hs_hi, srcp, dstp, zeros128)
        tc = _tc_layer_main_last if i == nl - 1 else _tc_layer_main
        h, hs_lo, hs_hi = tc(agg_lo, agg_hi, h, w_gcn[i],
                             b_gcn[i].reshape(1, -1), dv)
    hg_lo, hg_hi = _sc_pool_main(hs_lo, hs_hi, gidp)
    return hg_lo, hg_hi


def _solv_branch(x, edge_index, gids, w_emb, b_emb, w_gcn, b_gcn, ones128, zeros128):
    srcp, dstp = _pad_edges(edge_index, NW, NCHS)
    gidp = _pad_gids(gids)
    d0, d1 = _sc_deg(dstp, ones128, zeros128)
    h, hs_pad, dv = _tc_emb_solv(x, w_emb, b_emb.reshape(1, -1), d0, d1)
    nl = w_gcn.shape[0]
    for i in range(nl):
        a0, a1 = _sc_scatter_solv(hs_pad, srcp, dstp, zeros128)
        tc = _tc_layer_solv_last if i == nl - 1 else _tc_layer_solv
        h, hs_pad = tc(a0, a1, h, w_gcn[i], b_gcn[i].reshape(1, -1), dv)
    (hsg,) = _sc_pool_solv(hs_pad, gidp)
    return hsg


@jax.jit
def kernel(node_feats, edge_index, node_graph_ids, solv_node_feats,
           solv_edge_index, solv_node_graph_ids,
           W_emb, b_emb, W_emb_s, b_emb_s, W_gcn, b_gcn, W_gcn_s, b_gcn_s,
           W_l1, b_l1, W_l2, b_l2, W_l3a, b_l3a, W_l3b, b_l3b, W_out, b_out):
    ones128 = jnp.ones((CH, 128), jnp.float32)
    zeros128 = jnp.zeros((ACC_ROWS, 128), jnp.float32)
    hg_lo, hg_hi = _main_branch(
        node_feats, edge_index, node_graph_ids,
        W_emb, b_emb, W_gcn, b_gcn, ones128, zeros128)
    hsg = _solv_branch(
        solv_node_feats, solv_edge_index, solv_node_graph_ids,
        W_emb_s, b_emb_s, W_gcn_s, b_gcn_s, ones128, zeros128)
    return _tc_head(hg_lo, hg_hi, hsg,
                    W_l1, b_l1, W_l2, b_l2,
                    W_l3a, b_l3a, W_l3b, b_l3b, W_out, b_out)
